# Initial kernel scaffold; baseline (speedup 1.0000x reference)
import jax
import jax.numpy as jnp
from jax.experimental import pallas as pl


def kernel(x, table):
    raise NotImplementedError("write your pallas kernel here")


# speedup vs baseline: 154.5427x; 154.5427x over previous
import functools

import jax
import jax.numpy as jnp
import numpy as np
from jax import lax
from jax.experimental import pallas as pl
from jax.experimental.pallas import tpu as pltpu
from jax.experimental.pallas import tpu_sc as plsc

DIM = 3
NFEAT = 8
HASHMAP = 524288
MASK = HASHMAP - 1
RES = 128.0
P1 = int(np.uint32(2654435761).view(np.int32))
P2 = int(np.uint32(805459861).view(np.int32))

NC = 2
NS = 16
NW = NC * NS
L = 16

P = 128
NG = P // L

NBLK = HASHMAP // P
BPT = NBLK // NS
B4 = 4
NIT0 = BPT // B4


def _body(x3_hbm, t3_hbm, o3_hbm, scr_hbm,
          tbuf, rbufA, rbufB,
          xbufA, xbufB, idxA, idxB, wA, wB, rowsA, rowsB, outA, outB,
          semA, semB, semSA, semSB, *, npt):
    cid = lax.axis_index("c")
    sid = lax.axis_index("s")
    wid = sid * jnp.int32(NC) + cid
    nchunk = npt // P

    iota = lax.iota(jnp.int32, L)
    resf = jnp.float32(RES)
    one = jnp.float32(1.0)
    p1 = jnp.int32(P1)
    p2 = jnp.int32(P2)

    scr_base = cid * jnp.int32(HASHMAP)
    blk0 = sid * jnp.int32(BPT)

    def relayout(it, rbuf, semS):
        b = blk0 + it * jnp.int32(B4)
        pltpu.sync_copy(t3_hbm.at[pl.ds(b, B4)], tbuf)
        for bb in range(B4):
            for f in range(NFEAT):
                for g in range(NG):
                    s = g * L
                    v = tbuf[bb, f, pl.ds(s, L)]
                    rvec = jnp.int32(bb * P + s) + iota
                    plsc.store_scatter(
                        rbuf, [rvec, jnp.full((L,), f, jnp.int32)], v)
        row0 = scr_base + (b * jnp.int32(P))
        return pltpu.async_copy(rbuf, scr_hbm.at[pl.ds(row0, B4 * P)], semS)

    def relayout_wait(it, rbuf, semS):
        b = blk0 + it * jnp.int32(B4)
        row0 = scr_base + (b * jnp.int32(P))
        pltpu.make_async_copy(
            rbuf, scr_hbm.at[pl.ds(row0, B4 * P)], semS).wait()

    def p0_loop(ii, carry):
        it = ii * jnp.int32(2)

        @pl.when(ii > jnp.int32(0))
        def _():
            relayout_wait(it - jnp.int32(2), rbufA, semSA)

        relayout(it, rbufA, semSA)

        @pl.when(ii > jnp.int32(0))
        def _():
            relayout_wait(it - jnp.int32(1), rbufB, semSB)

        relayout(it + jnp.int32(1), rbufB, semSB)
        return carry

    nh0 = NIT0 // 2
    lax.fori_loop(jnp.int32(0), jnp.int32(nh0), p0_loop, jnp.int32(0))
    relayout_wait(jnp.int32(NIT0 - 2), rbufA, semSA)
    relayout_wait(jnp.int32(NIT0 - 1), rbufB, semSB)
    plsc.subcore_barrier()

    blk_base = wid * jnp.int32(npt // P)

    def compute(blk, xbuf, idx_buf, w_buf):
        pltpu.sync_copy(x3_hbm.at[blk], xbuf)
        for g in range(NG):
            s = g * L
            xs0 = xbuf[0, pl.ds(s, L)] * resf
            xs1 = xbuf[1, pl.ds(s, L)] * resf
            xs2 = xbuf[2, pl.ds(s, L)] * resf
            xi0 = xs0.astype(jnp.int32)
            xi1 = xs1.astype(jnp.int32)
            xi2 = xs2.astype(jnp.int32)
            b0 = xs0 - xi0.astype(jnp.float32)
            b1 = xs1 - xi1.astype(jnp.float32)
            b2 = xs2 - xi2.astype(jnp.float32)
            a0 = one - b0
            a1 = one - b1
            a2 = one - b2
            m1 = xi1 * p1
            m2 = xi2 * p2
            m1b = m1 + p1
            m2b = m2 + p2
            h0 = xi0
            h0b = xi0 + jnp.int32(1)
            t = [a0 * a1, b0 * a1, a0 * b1, b0 * b1]
            for c in range(8):
                hx = h0b if (c & 1) else h0
                hy = m1b if (c & 2) else m1
                hz = m2b if (c & 4) else m2
                h = ((hx ^ hy ^ hz) & jnp.int32(MASK)) + scr_base
                idx_buf[c, pl.ds(s, L)] = h
                w_buf[c, pl.ds(s, L)] = t[c & 3] * (b2 if (c & 4) else a2)

    def fire(idx_buf, rows_buf, sem):
        for c in range(8):
            pltpu.async_copy(scr_hbm.at[idx_buf.at[jnp.int32(c)]],
                             rows_buf.at[jnp.int32(c)], sem)

    def drain(idx_buf, rows_buf, sem):
        for c in range(8):
            pltpu.make_async_copy(scr_hbm.at[idx_buf.at[jnp.int32(c)]],
                                  rows_buf.at[jnp.int32(c)], sem).wait()

    def accum(blk, w_buf, rows_buf, out_buf):
        for g in range(NG):
            s = g * L
            pvec = iota + jnp.int32(s)
            accs = [jnp.zeros((L,), jnp.float32) for _ in range(NFEAT)]
            for c in range(8):
                wv = w_buf[c, pl.ds(s, L)]
                cvec = jnp.full((L,), c, jnp.int32)
                for f in range(NFEAT):
                    val = plsc.load_gather(
                        rows_buf, [cvec, pvec, jnp.full((L,), f, jnp.int32)])
                    accs[f] = accs[f] + wv * val
            for f in range(NFEAT):
                out_buf[f, pl.ds(s, L)] = accs[f]
        pltpu.sync_copy(out_buf, o3_hbm.at[blk])

    nhalf = nchunk // 2

    compute(blk_base, xbufA, idxA, wA)
    fire(idxA, rowsA, semA)

    def loop_body(jj, carry):
        j = jj * jnp.int32(2)
        compute(blk_base + j + jnp.int32(1), xbufB, idxB, wB)
        fire(idxB, rowsB, semB)
        drain(idxA, rowsA, semA)
        accum(blk_base + j, wA, rowsA, outA)

        @pl.when(jj < jnp.int32(nhalf - 1))
        def _():
            compute(blk_base + j + jnp.int32(2), xbufA, idxA, wA)
            fire(idxA, rowsA, semA)

        drain(idxB, rowsB, semB)
        accum(blk_base + j + jnp.int32(1), wB, rowsB, outB)
        return carry

    lax.fori_loop(jnp.int32(0), jnp.int32(nhalf), loop_body, jnp.int32(0))


@functools.partial(jax.jit, static_argnames=())
def _hash_grid(x, table):
    n = x.shape[0]
    npt = n // NW
    nblk = n // P
    t3 = jnp.transpose(jnp.reshape(table, (NBLK, P, NFEAT)), (0, 2, 1))
    xp = jnp.pad(x, ((0, 0), (0, 1)))
    x3 = jnp.transpose(jnp.reshape(xp, (nblk, P, 4)), (0, 2, 1))
    mesh = plsc.VectorSubcoreMesh(core_axis_name="c", subcore_axis_name="s")
    dbl = lambda t: [t, t]
    o3, _ = pl.kernel(
        functools.partial(_body, npt=npt),
        out_type=(jax.ShapeDtypeStruct((nblk, NFEAT, P), jnp.float32),
                  jax.ShapeDtypeStruct((NC * HASHMAP, NFEAT), jnp.float32)),
        mesh=mesh,
        compiler_params=pltpu.CompilerParams(
            needs_layout_passes=False, use_tc_tiling_on_sc=False),
        scratch_types=[
            pltpu.VMEM((B4, NFEAT, P), jnp.float32),
            *dbl(pltpu.VMEM((B4 * P, NFEAT), jnp.float32)),
            *dbl(pltpu.VMEM((4, P), jnp.float32)),
            *dbl(pltpu.VMEM((8, P), jnp.int32)),
            *dbl(pltpu.VMEM((8, P), jnp.float32)),
            *dbl(pltpu.VMEM((8, P, NFEAT), jnp.float32)),
            *dbl(pltpu.VMEM((NFEAT, P), jnp.float32)),
            *dbl(pltpu.SemaphoreType.DMA),
            *dbl(pltpu.SemaphoreType.DMA),
        ],
    )(x3, t3)
    return jnp.reshape(jnp.transpose(o3, (0, 2, 1)), (n, NFEAT))


def kernel(x, table):
    return _hash_grid(x.astype(jnp.float32), table.astype(jnp.float32))
